# Initial kernel scaffold; baseline (speedup 1.0000x reference)
import jax
import jax.numpy as jnp
from jax.experimental import pallas as pl


def kernel(x, edge_index_r0, edge_index_r1, edge_index_r2, W0, b0, W1, b1, W2, b2, h_bias):
    raise NotImplementedError("write your pallas kernel here")


# speedup vs baseline: 2.3670x; 2.3670x over previous
import dataclasses
import functools

import jax
import jax.numpy as jnp
from jax import lax
from jax.experimental import pallas as pl
from jax.experimental.pallas import tpu as pltpu
from jax.experimental.pallas import tpu_sc as plsc

D = 128
NUM_CHUNKS = 4
N_SC = 2
N_TILES = 16
G = 128


def _matmul(xp, W, b):
    rows = xp.shape[0]

    def body(x_ref, w_ref, b_ref, o_ref):
        o_ref[...] = (
            jnp.dot(x_ref[...], w_ref[...], preferred_element_type=jnp.float32)
            + b_ref[...]
        )

    return pl.pallas_call(
        body,
        grid=(rows // 256,),
        in_specs=[
            pl.BlockSpec((256, D), lambda i: (i, 0)),
            pl.BlockSpec((D, D), lambda i: (0, 0)),
            pl.BlockSpec((1, D), lambda i: (0, 0)),
        ],
        out_specs=pl.BlockSpec((256, D), lambda i: (i, 0)),
        out_shape=jax.ShapeDtypeStruct((rows, D), jnp.float32),
    )(xp, W, b.reshape(1, D))


def _combine(c0, c1, c2, hb):
    rows = c0.shape[0]

    def body(a_ref, b_ref, c_ref, hb_ref, o_ref):
        h = jnp.maximum(a_ref[...], 0.0)
        h = jnp.maximum(h + b_ref[...], 0.0)
        h = jnp.maximum(h + c_ref[...], 0.0)
        o_ref[...] = h + hb_ref[...]

    return pl.pallas_call(
        body,
        grid=(rows // 512,),
        in_specs=[
            pl.BlockSpec((512, D), lambda i: (i, 0)),
            pl.BlockSpec((512, D), lambda i: (i, 0)),
            pl.BlockSpec((512, D), lambda i: (i, 0)),
            pl.BlockSpec((1, D), lambda i: (0, 0)),
        ],
        out_specs=pl.BlockSpec((512, D), lambda i: (i, 0)),
        out_shape=jax.ShapeDtypeStruct((rows, D), jnp.float32),
    )(c0, c1, c2, hb.reshape(1, D))


def _sc_segment_sum(m, src, dst, ch):
    rows = m.shape[0]
    pe = src.shape[0]
    ept = pe // N_TILES
    ew = 1792
    nw = ept // ew
    nbuf = ew // G + 1
    rpt = ch // N_TILES
    zr = rpt // 8
    trash = ch
    assert ept % ew == 0 and rpt % 8 == 0 and zr <= G

    mesh = plsc.VectorSubcoreMesh(core_axis_name="c", subcore_axis_name="s")
    cp = pltpu.CompilerParams()
    if "needs_layout_passes" in pltpu.CompilerParams.__dataclass_fields__:
        cp = dataclasses.replace(cp, needs_layout_passes=False)

    @functools.partial(
        pl.kernel,
        out_type=jax.ShapeDtypeStruct((rows, D), jnp.float32),
        mesh=mesh,
        compiler_params=cp,
        scratch_types=[
            pltpu.VMEM((ew,), jnp.int32),
            pltpu.VMEM((ew,), jnp.int32),
            pltpu.VMEM((nbuf, G), jnp.int32),
            pltpu.VMEM((nbuf, G), jnp.int32),
            pltpu.VMEM((G, D), jnp.float32),
            pltpu.VMEM_SHARED((ch + 8, D), jnp.float32),
            pltpu.SemaphoreType.DMA,
        ],
    )
    def k(m_hbm, src_hbm, dst_hbm, out_hbm, draw, sraw, srcc, dstc, rows_v,
          acc, sem):
        core = lax.axis_index("c")
        tid = lax.axis_index("s")
        zer = jnp.zeros((16,), jnp.int32)
        trv = jnp.full((16,), trash, jnp.int32)

        def gs_body(g, carry):
            pltpu.async_copy(m_hbm.at[srcc.at[g]], rows_v, sem).wait()
            pltpu.sync_copy(rows_v, acc.at[dstc.at[g]], add=True)
            return carry

        for ci in range(NUM_CHUNKS // N_SC):
            base = (N_SC * core + ci) * ch

            @pl.loop(0, G)
            def _(r):
                for c in range(D // 16):
                    rows_v[r, pl.ds(c * 16, 16)] = jnp.zeros((16,),
                                                             jnp.float32)

            @pl.loop(0, 8)
            def _(z):
                pltpu.sync_copy(rows_v.at[pl.ds(0, zr)],
                                acc.at[pl.ds(tid * rpt + z * zr, zr)])

            plsc.subcore_barrier()

            def win_body(w, cnt):
                e0 = tid * ept + w * ew
                pltpu.sync_copy(dst_hbm.at[pl.ds(e0, ew)], draw)
                pltpu.sync_copy(src_hbm.at[pl.ds(e0, ew)], sraw)

                def scan_body(j, cnt):
                    dstv = draw[pl.ds(j * 16, 16)]
                    srcv = sraw[pl.ds(j * 16, 16)]
                    mask = jnp.logical_and(dstv >= base, dstv < base + ch)
                    mi = mask.astype(jnp.int32)
                    pos = cnt + plsc.cumsum(mi) - 1
                    r = lax.shift_right_logical(pos, 7)
                    cc = lax.bitwise_and(pos, G - 1)
                    plsc.store_scatter(srcc, [r, cc], srcv, mask=mask)
                    plsc.store_scatter(dstc, [r, cc], dstv - base, mask=mask)
                    return cnt + jnp.sum(mi)

                cnt = lax.fori_loop(0, ew // 16, scan_body, cnt, unroll=False)

                nfull = lax.shift_right_logical(cnt, 7)
                lax.fori_loop(0, nfull, gs_body, jnp.int32(0), unroll=False)

                nfv = jnp.full((16,), nfull, jnp.int32)
                for c in range(G // 16):
                    lanes = c * 16 + lax.iota(jnp.int32, 16)
                    sv = plsc.load_gather(srcc, [nfv, lanes])
                    dv = plsc.load_gather(dstc, [nfv, lanes])
                    srcc[0, pl.ds(c * 16, 16)] = sv
                    dstc[0, pl.ds(c * 16, 16)] = dv
                return lax.bitwise_and(cnt, G - 1)

            cnt = lax.fori_loop(0, nw, win_body, jnp.int32(0), unroll=False)

            for c in range(G // 16):
                lanes = c * 16 + lax.iota(jnp.int32, 16)
                pm = lanes >= cnt
                plsc.store_scatter(srcc, [zer, lanes], zer, mask=pm)
                plsc.store_scatter(dstc, [zer, lanes], trv, mask=pm)
            nb = lax.shift_right_logical(cnt + (G - 1), 7)
            lax.fori_loop(0, nb, gs_body, jnp.int32(0), unroll=False)

            plsc.subcore_barrier()
            pltpu.sync_copy(
                acc.at[pl.ds(tid * rpt, rpt)],
                out_hbm.at[pl.ds(base + tid * rpt, rpt)],
            )

    return k(m, src, dst)


def _pad_edges(ei, pe):
    pad = pe - ei.shape[1]
    src = jnp.concatenate([ei[0], jnp.zeros((pad,), jnp.int32)])
    dst = jnp.concatenate([ei[1], jnp.full((pad,), -1, jnp.int32)])
    return src, dst


def kernel(x, edge_index_r0, edge_index_r1, edge_index_r2,
           W0, b0, W1, b1, W2, b2, h_bias):
    n = x.shape[0]
    e = edge_index_r0.shape[1]

    ch = -(-n // (NUM_CHUNKS * N_TILES * 8)) * (N_TILES * 8)
    rows = NUM_CHUNKS * ch
    assert rows % 512 == 0 and ch * (D * 4) <= 8 * 2**20 - 8 * D * 4
    ept = -(-e // (N_TILES * G)) * G
    pe = ept * N_TILES

    xp = jnp.concatenate([x, jnp.zeros((rows - n, D), x.dtype)])

    contribs = []
    for ei, W, b in ((edge_index_r0, W0, b0),
                     (edge_index_r1, W1, b1),
                     (edge_index_r2, W2, b2)):
        m = _matmul(xp, W, b)
        src, dst = _pad_edges(ei, pe)
        contribs.append(_sc_segment_sum(m, src, dst, ch))

    hs = _combine(contribs[0], contribs[1], contribs[2], h_bias)
    return hs[:n]
